# baseline (speedup 1.0000x reference)
import functools

import jax
import jax.numpy as jnp
from jax import lax
from jax.experimental import pallas as pl
from jax.experimental.pallas import tpu as pltpu
from jax.experimental.pallas import tpu_sc as plsc

BATCH = 16384
EMBED_DIM = 32
PACK = 128 // EMBED_DIM

_NC = 2
_NS = 16
_NW = _NC * _NS
_ROWS_PER_W = BATCH // _NW
_CHUNK = 128
_CHUNKS_PER_W = _ROWS_PER_W // _CHUNK
_L = 16

_PANEL = 4096
_NPANEL = 62
_QUARTER = _PANEL * _NPANEL


def _repack_body(inT_ref, out_ref):
    a = pl.program_id(1)
    eye = (lax.broadcasted_iota(jnp.int32, (EMBED_DIM, EMBED_DIM), 0) ==
           lax.broadcasted_iota(jnp.int32, (EMBED_DIM, EMBED_DIM), 1)
           ).astype(jnp.float32)
    t = lax.dot_general(inT_ref[...], eye, (((0,), (0,)), ((), ())),
                        preferred_element_type=jnp.float32,
                        precision=lax.Precision.HIGHEST)
    for k in range(PACK):
        @pl.when(a == k)
        def _():
            out_ref[:, k * EMBED_DIM:(k + 1) * EMBED_DIM] = t


def _tc_repack(tableT):
    grid = (_NPANEL, PACK)
    last_block = pl.cdiv(tableT.shape[1], _PANEL) - 1
    return pl.pallas_call(
        _repack_body,
        grid=grid,
        in_specs=[pl.BlockSpec(
            (EMBED_DIM, _PANEL),
            lambda i, a: (0, jnp.minimum(a * _NPANEL + i, last_block)))],
        out_specs=pl.BlockSpec((_PANEL, 128), lambda i, a: (i, 0)),
        out_shape=jax.ShapeDtypeStruct((_QUARTER, 128), jnp.float32),
    )(tableT)


def _sc_gather(table4, idx2d, quarter):
    mesh = plsc.VectorSubcoreMesh(core_axis_name="c", subcore_axis_name="s")

    @functools.partial(
        pl.kernel,
        mesh=mesh,
        out_type=jax.ShapeDtypeStruct((BATCH, 128), jnp.float32),
        scratch_types=[
            pltpu.VMEM((_CHUNKS_PER_W, _CHUNK), jnp.int32),
            pltpu.VMEM((_CHUNKS_PER_W, _CHUNK), jnp.int32),
            pltpu.VMEM((_ROWS_PER_W, 128), jnp.float32),
            pltpu.SemaphoreType.DMA,
        ],
    )
    def gather(table_hbm, idx_hbm, out_hbm, idx_v, idxq_v, rows_v, sem):
        wid = lax.axis_index("s") * _NC + lax.axis_index("c")
        pltpu.sync_copy(idx_hbm.at[pl.ds(wid * _CHUNKS_PER_W, _CHUNKS_PER_W)],
                        idx_v)
        for j in range(_CHUNKS_PER_W):
            for k in range(_CHUNK // _L):
                idxq_v[j, pl.ds(k * _L, _L)] = lax.rem(
                    idx_v[j, pl.ds(k * _L, _L)], quarter)
        copies = [
            pltpu.async_copy(table_hbm.at[idxq_v.at[j]],
                             rows_v.at[pl.ds(j * _CHUNK, _CHUNK)], sem)
            for j in range(_CHUNKS_PER_W)
        ]
        for c in copies:
            c.wait()
        pltpu.sync_copy(rows_v, out_hbm.at[pl.ds(wid * _ROWS_PER_W,
                                                 _ROWS_PER_W)])

    return gather(table4, idx2d)


_BB = 2048


def _dot(a, b):
    return jnp.dot(a, b, preferred_element_type=jnp.float32,
                   precision=lax.Precision.HIGHEST)


def _mlp_body(g_ref, idx_ref, num_ref, w1a_ref, w1b_ref, b1_ref, w2_ref,
              b2_ref, out_ref):
    off = idx_ref[...] // _QUARTER
    g = g_ref[...]
    emb = jnp.where(off == 0, g[:, 0:32],
          jnp.where(off == 1, g[:, 32:64],
          jnp.where(off == 2, g[:, 64:96], g[:, 96:128])))
    h = _dot(emb, w1a_ref[...]) + _dot(num_ref[...], w1b_ref[...])
    h = jnp.maximum(h + b1_ref[...], 0.0)
    out_ref[...] = _dot(h, w2_ref[...]) + b2_ref[...]


def _tc_mlp(g, idx, num, w1a, w1b, b1, w2, b2):
    grid = (BATCH // _BB,)
    return pl.pallas_call(
        _mlp_body,
        grid=grid,
        in_specs=[
            pl.BlockSpec((_BB, 128), lambda i: (i, 0)),
            pl.BlockSpec((_BB, 1), lambda i: (i, 0)),
            pl.BlockSpec((_BB, num.shape[1]), lambda i: (i, 0)),
            pl.BlockSpec(w1a.shape, lambda i: (0, 0)),
            pl.BlockSpec(w1b.shape, lambda i: (0, 0)),
            pl.BlockSpec(b1.shape, lambda i: (0, 0)),
            pl.BlockSpec(w2.shape, lambda i: (0, 0)),
            pl.BlockSpec(b2.shape, lambda i: (0, 0)),
        ],
        out_specs=pl.BlockSpec((_BB, EMBED_DIM), lambda i: (i, 0)),
        out_shape=jax.ShapeDtypeStruct((BATCH, EMBED_DIM), jnp.float32),
    )(g, idx, num, w1a, w1b, b1, w2, b2)


def kernel(user_idx, numerical_features, user_embed, W1, b1, W2, b2):
    idx = user_idx.astype(jnp.int32)
    idx2d = idx.reshape(BATCH // _CHUNK, _CHUNK)
    table4 = _tc_repack(user_embed.T)
    g = _sc_gather(table4, idx2d, _QUARTER)
    return _tc_mlp(g, idx, numerical_features,
                   W1[:EMBED_DIM], W1[EMBED_DIM:],
                   b1.reshape(1, -1), W2, b2.reshape(1, -1))


# speedup vs baseline: 1.9749x; 1.9749x over previous
import functools

import jax
import jax.numpy as jnp
from jax import lax
from jax.experimental import pallas as pl
from jax.experimental.pallas import tpu as pltpu
from jax.experimental.pallas import tpu_sc as plsc

BATCH = 16384
EMBED_DIM = 32

_NC = 2
_NS = 16
_NW = _NC * _NS
_ROWS_PER_W = BATCH // _NW
_CHUNK = 64
_CHUNKS_PER_W = _ROWS_PER_W // _CHUNK
_L = 16


def _sc_gather(table, idx2d):
    mesh = plsc.VectorSubcoreMesh(core_axis_name="c", subcore_axis_name="s")

    @functools.partial(
        pl.kernel,
        mesh=mesh,
        compiler_params=pltpu.CompilerParams(needs_layout_passes=False),
        out_type=jax.ShapeDtypeStruct((BATCH, EMBED_DIM), jnp.float32),
        scratch_types=[
            pltpu.VMEM((_CHUNKS_PER_W, _CHUNK), jnp.int32),
            pltpu.VMEM((_CHUNK, EMBED_DIM), jnp.float32),
            pltpu.SemaphoreType.DMA,
        ],
    )
    def gather(table_hbm, idx_hbm, out_hbm, idx_v, rows_v, sem):
        wid = lax.axis_index("s") * _NC + lax.axis_index("c")
        lanes = lax.iota(jnp.int32, _L)
        pltpu.sync_copy(idx_hbm.at[pl.ds(wid * _CHUNKS_PER_W, _CHUNKS_PER_W)],
                        idx_v)
        for j in range(_CHUNKS_PER_W):
            copies = []
            for g in range(_CHUNK // _L):
                v16 = idx_v[j, pl.ds(g * _L, _L)]
                for t in range(_L):
                    r = jnp.sum(jnp.where(lanes == t, v16, 0))
                    copies.append(pltpu.async_copy(
                        table_hbm.at[pl.ds(r, 1)],
                        rows_v.at[pl.ds(g * _L + t, 1)], sem))
            for c in copies:
                c.wait()
            pltpu.sync_copy(
                rows_v,
                out_hbm.at[pl.ds(wid * _ROWS_PER_W + j * _CHUNK, _CHUNK)])

    return gather(table, idx2d)


_BB = 2048


def _dot(a, b):
    return jnp.dot(a, b, preferred_element_type=jnp.float32)


def _mlp_body(emb_ref, num_ref, w1a_ref, w1b_ref, b1_ref, w2_ref, b2_ref,
              out_ref):
    h = _dot(emb_ref[...], w1a_ref[...]) + _dot(num_ref[...], w1b_ref[...])
    h = jnp.maximum(h + b1_ref[...], 0.0)
    out_ref[...] = _dot(h, w2_ref[...]) + b2_ref[...]


def _tc_mlp(emb, num, w1a, w1b, b1, w2, b2):
    grid = (BATCH // _BB,)
    return pl.pallas_call(
        _mlp_body,
        grid=grid,
        in_specs=[
            pl.BlockSpec((_BB, EMBED_DIM), lambda i: (i, 0)),
            pl.BlockSpec((_BB, num.shape[1]), lambda i: (i, 0)),
            pl.BlockSpec(w1a.shape, lambda i: (0, 0)),
            pl.BlockSpec(w1b.shape, lambda i: (0, 0)),
            pl.BlockSpec(b1.shape, lambda i: (0, 0)),
            pl.BlockSpec(w2.shape, lambda i: (0, 0)),
            pl.BlockSpec(b2.shape, lambda i: (0, 0)),
        ],
        out_specs=pl.BlockSpec((_BB, EMBED_DIM), lambda i: (i, 0)),
        out_shape=jax.ShapeDtypeStruct((BATCH, EMBED_DIM), jnp.float32),
    )(emb, num, w1a, w1b, b1, w2, b2)


def kernel(user_idx, numerical_features, user_embed, W1, b1, W2, b2):
    idx = user_idx.astype(jnp.int32)
    idx2d = idx.reshape(BATCH // _CHUNK, _CHUNK)
    emb = _sc_gather(user_embed, idx2d)
    return _tc_mlp(emb, numerical_features,
                   W1[:EMBED_DIM], W1[EMBED_DIM:],
                   b1.reshape(1, -1), W2, b2.reshape(1, -1))


# speedup vs baseline: 2.0327x; 1.0293x over previous
import functools

import jax
import jax.numpy as jnp
from jax import lax
from jax.experimental import pallas as pl
from jax.experimental.pallas import tpu as pltpu
from jax.experimental.pallas import tpu_sc as plsc

BATCH = 16384
EMBED_DIM = 32

_NC = 2
_NS = 16
_NW = _NC * _NS
_ROWS_PER_W = BATCH // _NW
_CHUNK = 64
_CHUNKS_PER_W = _ROWS_PER_W // _CHUNK
_L = 16


def _sc_gather(table, idx2d):
    mesh = plsc.VectorSubcoreMesh(core_axis_name="c", subcore_axis_name="s")

    @functools.partial(
        pl.kernel,
        mesh=mesh,
        compiler_params=pltpu.CompilerParams(needs_layout_passes=False),
        out_type=jax.ShapeDtypeStruct((BATCH, EMBED_DIM), jnp.float32),
        scratch_types=[
            pltpu.VMEM((_CHUNKS_PER_W, _CHUNK), jnp.int32),
            pltpu.VMEM((_CHUNK, EMBED_DIM), jnp.float32),
            pltpu.SemaphoreType.DMA,
        ],
    )
    def gather(table_hbm, idx_hbm, out_hbm, idx_v, rows_v, sem):
        wid = lax.axis_index("s") * _NC + lax.axis_index("c")
        lanes = lax.iota(jnp.int32, _L)
        pltpu.sync_copy(idx_hbm.at[pl.ds(wid * _CHUNKS_PER_W, _CHUNKS_PER_W)],
                        idx_v)
        for j in range(_CHUNKS_PER_W):
            copies = []
            for g in range(_CHUNK // _L):
                v16 = idx_v[j, pl.ds(g * _L, _L)]
                for t in range(_L):
                    r = jnp.sum(jnp.where(lanes == t, v16, 0))
                    copies.append(pltpu.async_copy(
                        table_hbm.at[pl.ds(r, 1)],
                        rows_v.at[pl.ds(g * _L + t, 1)], sem))
            for c in copies:
                c.wait()
            pltpu.sync_copy(
                rows_v,
                out_hbm.at[pl.ds(wid * _ROWS_PER_W + j * _CHUNK, _CHUNK)])

    return gather(table, idx2d)


_BB = 2048


def _dot(a, b):
    return jnp.dot(a, b, preferred_element_type=jnp.float32)


def _mlp_body(emb_ref, numT_ref, w1a_ref, w1b_ref, b1_ref, w2_ref, b2T_ref,
              outT_ref):
    h = _dot(emb_ref[...], w1a_ref[...])
    h = h + lax.dot_general(numT_ref[...], w1b_ref[...],
                            (((0,), (0,)), ((), ())),
                            preferred_element_type=jnp.float32)
    h = jnp.maximum(h + b1_ref[...], 0.0)
    outT_ref[...] = lax.dot_general(w2_ref[...], h,
                                    (((0,), (1,)), ((), ())),
                                    preferred_element_type=jnp.float32
                                    ) + b2T_ref[...]


def _tc_mlp(emb, numT, w1a, w1b, b1, w2, b2T):
    grid = (BATCH // _BB,)
    return pl.pallas_call(
        _mlp_body,
        grid=grid,
        in_specs=[
            pl.BlockSpec((_BB, EMBED_DIM), lambda i: (i, 0)),
            pl.BlockSpec((numT.shape[0], _BB), lambda i: (0, i)),
            pl.BlockSpec(w1a.shape, lambda i: (0, 0)),
            pl.BlockSpec(w1b.shape, lambda i: (0, 0)),
            pl.BlockSpec(b1.shape, lambda i: (0, 0)),
            pl.BlockSpec(w2.shape, lambda i: (0, 0)),
            pl.BlockSpec(b2T.shape, lambda i: (0, 0)),
        ],
        out_specs=pl.BlockSpec((EMBED_DIM, _BB), lambda i: (0, i)),
        out_shape=jax.ShapeDtypeStruct((EMBED_DIM, BATCH), jnp.float32),
    )(emb, numT, w1a, w1b, b1, w2, b2T)


def kernel(user_idx, numerical_features, user_embed, W1, b1, W2, b2):
    idx = user_idx.astype(jnp.int32)
    idx2d = idx.reshape(BATCH // _CHUNK, _CHUNK)
    emb = _sc_gather(user_embed, idx2d)
    outT = _tc_mlp(emb, numerical_features.T,
                   W1[:EMBED_DIM], W1[EMBED_DIM:],
                   b1.reshape(1, -1), W2, b2.reshape(-1, 1))
    return outT.T
